# Initial kernel scaffold; baseline (speedup 1.0000x reference)
import jax
import jax.numpy as jnp
from jax.experimental import pallas as pl


def kernel(x, table):
    raise NotImplementedError("write your pallas kernel here")


# speedup vs baseline: 2.2869x; 2.2869x over previous
import math

import jax
import jax.numpy as jnp
import numpy as np
from jax import lax
from jax.experimental import pallas as pl
from jax.experimental.pallas import tpu as pltpu
from jax.experimental.pallas import tpu_sc as plsc

V = 100000
D = 64
B = 4096
S = 200

NC = 2
NS = 16
NW = NC * NS

CHUNK = 128
ROWS_PER_W = (B * S) // NW
NCHUNK = ROWS_PER_W // CHUNK
SCALE = math.sqrt(D)


def _positional_encoding_doubled():
    position = np.arange(0, S, dtype=np.float32)[:, None]
    div_term = np.exp(
        np.arange(0, D, 2, dtype=np.float32) * -(math.log(10000.0) / D))
    enc = np.zeros((S, D), dtype=np.float32)
    enc[:, 0::2] = np.sin(position * div_term)
    enc[:, 1::2] = np.cos(position * div_term)
    return np.concatenate([enc, enc], axis=0)


_ENC2 = _positional_encoding_doubled()

_mesh = plsc.VectorSubcoreMesh(
    core_axis_name="c", subcore_axis_name="s", num_cores=NC, num_subcores=NS)


@jax.jit
def _emb_kernel(x3, table, enc2):
    @pl.kernel(
        out_type=jax.ShapeDtypeStruct((B * S, D), jnp.float32),
        mesh=_mesh,
        scratch_types=[
            pltpu.VMEM((NCHUNK, CHUNK), jnp.int32),
            pltpu.VMEM((2 * S, D), jnp.float32),
            pltpu.VMEM((CHUNK, D), jnp.float32),
            pltpu.SemaphoreType.DMA,
        ],
        compiler_params=pltpu.CompilerParams(use_tc_tiling_on_sc=False),
    )
    def body(x_hbm, table_hbm, enc_hbm, out_hbm, idx_v, enc_v, rows_v, sem):
        wid = lax.axis_index("s") * NC + lax.axis_index("c")
        pltpu.sync_copy(x_hbm.at[wid], idx_v)
        pltpu.sync_copy(enc_hbm, enc_v)

        def chunk_body(c, carry):
            base = lax.rem(c * CHUNK, S)
            pltpu.async_copy(table_hbm.at[idx_v.at[c]], rows_v, sem).wait()

            def row_body(r, carry2):
                e = base + r
                for cc in range(D // 16):
                    sl = pl.ds(cc * 16, 16)
                    rows_v[r, sl] = rows_v[r, sl] * SCALE + enc_v[e, sl]
                return carry2

            lax.fori_loop(0, CHUNK, row_body, 0, unroll=2)
            start = wid * ROWS_PER_W + c * CHUNK
            pltpu.sync_copy(rows_v, out_hbm.at[pl.ds(start, CHUNK)])
            return carry

        lax.fori_loop(0, NCHUNK, chunk_body, 0)

    return body(x3, table, enc2)


def kernel(x, table):
    x3 = x.reshape(NW, NCHUNK, CHUNK)
    out = _emb_kernel(x3, table, _ENC2)
    return out.reshape(B, S, D)


# speedup vs baseline: 2.6767x; 1.1704x over previous
import math

import jax
import jax.numpy as jnp
import numpy as np
from jax import lax
from jax.experimental import pallas as pl
from jax.experimental.pallas import tpu as pltpu
from jax.experimental.pallas import tpu_sc as plsc

V = 100000
D = 64
B = 4096
S = 200

NC = 2
NS = 16
NW = NC * NS

CHUNK = 128
ROWS_PER_W = (B * S) // NW
NCHUNK = ROWS_PER_W // CHUNK
SCALE = math.sqrt(D)


def _positional_encoding_doubled():
    position = np.arange(0, S, dtype=np.float32)[:, None]
    div_term = np.exp(
        np.arange(0, D, 2, dtype=np.float32) * -(math.log(10000.0) / D))
    enc = np.zeros((S, D), dtype=np.float32)
    enc[:, 0::2] = np.sin(position * div_term)
    enc[:, 1::2] = np.cos(position * div_term)
    return np.concatenate([enc, enc], axis=0)


_ENC2 = _positional_encoding_doubled()

_mesh = plsc.VectorSubcoreMesh(
    core_axis_name="c", subcore_axis_name="s", num_cores=NC, num_subcores=NS)


NB = 4
NGROUP = NCHUNK // NB


@jax.jit
def _emb_kernel(x3, table, enc2):
    @pl.kernel(
        out_type=jax.ShapeDtypeStruct((B * S, D), jnp.float32),
        mesh=_mesh,
        scratch_types=[
            pltpu.VMEM((NCHUNK, CHUNK), jnp.int32),
            pltpu.VMEM((2 * S, D), jnp.float32),
            pltpu.VMEM((NB, CHUNK, D), jnp.float32),
            pltpu.VMEM((NB, CHUNK, D), jnp.float32),
            pltpu.SemaphoreType.DMA((NB,)),
            pltpu.SemaphoreType.DMA((NB,)),
        ],
        compiler_params=pltpu.CompilerParams(use_tc_tiling_on_sc=False),
    )
    def body(x_hbm, table_hbm, enc_hbm, out_hbm,
             idx_v, enc_v, gbuf, obuf, gsem, osem):
        wid = lax.axis_index("s") * NC + lax.axis_index("c")
        pltpu.sync_copy(x_hbm.at[wid], idx_v)
        pltpu.sync_copy(enc_hbm, enc_v)

        for b in range(NB):
            pltpu.async_copy(table_hbm.at[idx_v.at[b]], gbuf.at[b], gsem.at[b])

        def group_body(g, carry):
            for b in range(NB):
                c = g * NB + b
                base = lax.rem(c * CHUNK, S)
                pltpu.make_async_copy(
                    table_hbm.at[idx_v.at[c]], gbuf.at[b], gsem.at[b]).wait()

                @pl.when(g > 0)
                def _():
                    pltpu.make_async_copy(
                        obuf.at[b], out_hbm.at[pl.ds(0, CHUNK)],
                        osem.at[b]).wait()

                def row_body(r, carry2):
                    e = base + r
                    for cc in range(D // 16):
                        sl = pl.ds(cc * 16, 16)
                        obuf[b, r, sl] = gbuf[b, r, sl] * SCALE + enc_v[e, sl]
                    return carry2

                lax.fori_loop(0, CHUNK, row_body, 0, unroll=2)

                cn = c + NB

                @pl.when(cn < NCHUNK)
                def _():
                    pltpu.async_copy(
                        table_hbm.at[idx_v.at[cn]], gbuf.at[b], gsem.at[b])

                start = wid * ROWS_PER_W + c * CHUNK
                pltpu.async_copy(
                    obuf.at[b], out_hbm.at[pl.ds(start, CHUNK)], osem.at[b])
            return carry

        lax.fori_loop(0, NGROUP, group_body, 0)

        for b in range(NB):
            pltpu.make_async_copy(
                obuf.at[b], out_hbm.at[pl.ds(0, CHUNK)], osem.at[b]).wait()

    return body(x3, table, enc2)


def kernel(x, table):
    x3 = x.reshape(NW, NCHUNK, CHUNK)
    out = _emb_kernel(x3, table, _ENC2)
    return out.reshape(B, S, D)


# speedup vs baseline: 3.2728x; 1.2227x over previous
import math

import jax
import jax.numpy as jnp
import numpy as np
from jax import lax
from jax.experimental import pallas as pl
from jax.experimental.pallas import tpu as pltpu
from jax.experimental.pallas import tpu_sc as plsc

V = 100000
D = 64
B = 4096
S = 200

NC = 2
NS = 16
NW = NC * NS
RPW = B // NW
IDX_CHUNK = 40
NIC = S // IDX_CHUNK
NBG = 4
NBO = 2
NGROUP = RPW // NBG
SCALE = math.sqrt(D)


def _positional_encoding():
    position = np.arange(0, S, dtype=np.float32)[:, None]
    div_term = np.exp(
        np.arange(0, D, 2, dtype=np.float32) * -(math.log(10000.0) / D))
    enc = np.zeros((S, D), dtype=np.float32)
    enc[:, 0::2] = np.sin(position * div_term)
    enc[:, 1::2] = np.cos(position * div_term)
    return enc


_ENC = _positional_encoding()

_mesh = plsc.VectorSubcoreMesh(
    core_axis_name="c", subcore_axis_name="s", num_cores=NC, num_subcores=NS)


@jax.jit
def _emb_kernel(x, table, enc):
    @pl.kernel(
        out_type=jax.ShapeDtypeStruct((B, S, D), jnp.float32),
        mesh=_mesh,
        scratch_types=[
            pltpu.VMEM((RPW, S), jnp.int32),
            pltpu.VMEM((S, D), jnp.float32),
            pltpu.VMEM((NBG, S, D), jnp.float32),
            pltpu.VMEM((NBO, S, D), jnp.float32),
            pltpu.SemaphoreType.DMA((NBG,)),
            pltpu.SemaphoreType.DMA((NBO,)),
        ],
        compiler_params=pltpu.CompilerParams(use_tc_tiling_on_sc=False),
    )
    def body(x_hbm, table_hbm, enc_hbm, out_hbm,
             idx_v, enc_v, gbuf, obuf, gsem, osem):
        wid = lax.axis_index("s") * NC + lax.axis_index("c")
        row0 = wid * RPW
        pltpu.sync_copy(x_hbm.at[pl.ds(row0, RPW)], idx_v)
        pltpu.sync_copy(enc_hbm, enc_v)

        def issue_gathers(r, b):
            for j in range(NIC):
                sl = pl.ds(j * IDX_CHUNK, IDX_CHUNK)
                pltpu.async_copy(
                    table_hbm.at[idx_v.at[r, sl]], gbuf.at[b, sl], gsem.at[b])

        def wait_gathers(r, b):
            for j in range(NIC):
                sl = pl.ds(j * IDX_CHUNK, IDX_CHUNK)
                pltpu.make_async_copy(
                    table_hbm.at[idx_v.at[r, sl]], gbuf.at[b, sl],
                    gsem.at[b]).wait()

        for b in range(NBG):
            issue_gathers(b, b)

        def group_body(g, carry):
            for b in range(NBG):
                r = g * NBG + b
                o = b % NBO
                wait_gathers(r, b)

                if b < NBO:
                    @pl.when(g > 0)
                    def _():
                        pltpu.make_async_copy(
                            obuf.at[o], out_hbm.at[0], osem.at[o]).wait()
                else:
                    pltpu.make_async_copy(
                        obuf.at[o], out_hbm.at[0], osem.at[o]).wait()

                def s_body(s, carry2):
                    for cc in range(D // 16):
                        sl = pl.ds(cc * 16, 16)
                        obuf[o, s, sl] = gbuf[b, s, sl] * SCALE + enc_v[s, sl]
                    return carry2

                lax.fori_loop(0, S, s_body, 0, unroll=2)

                @pl.when(g < NGROUP - 1)
                def _():
                    issue_gathers(r + NBG, b)

                pltpu.async_copy(obuf.at[o], out_hbm.at[row0 + r], osem.at[o])
            return carry

        lax.fori_loop(0, NGROUP, group_body, 0)

        for o in range(NBO):
            pltpu.make_async_copy(
                obuf.at[o], out_hbm.at[0], osem.at[o]).wait()

    return body(x, table, enc)


def kernel(x, table):
    return _emb_kernel(x, table, _ENC)
